# baseline (speedup 1.0000x reference)
import functools

import jax
import jax.numpy as jnp
from jax.experimental import pallas as pl
from jax.experimental.pallas import tpu as pltpu

_TM = 400


def _gcn_kernel(nblocks, adj_any, x_ref, w_ref, b_ref, out_any, supp_ref):
    supp_ref[...] = jnp.dot(x_ref[...], w_ref[...], preferred_element_type=jnp.float32)
    bias_row = b_ref[...]

    def inner(adj_blk, out_blk):
        acc = jnp.dot(adj_blk[...], supp_ref[...], preferred_element_type=jnp.float32)
        out_blk[...] = jax.nn.sigmoid(acc + bias_row)

    n = adj_any.shape[1]
    out_f = supp_ref.shape[1]
    pipeline = pltpu.emit_pipeline(
        inner,
        grid=(nblocks,),
        in_specs=[pl.BlockSpec((_TM, n), lambda i: (i, 0))],
        out_specs=[pl.BlockSpec((_TM, out_f), lambda i: (i, 0))],
    )
    pipeline(adj_any, out_any)


def kernel(input, adj, weight, bias):
    n, in_f = input.shape
    out_f = weight.shape[1]
    bias2d = bias.reshape(1, out_f)
    nblocks = n // _TM
    return pl.pallas_call(
        functools.partial(_gcn_kernel, nblocks),
        in_specs=[
            pl.BlockSpec(memory_space=pltpu.MemorySpace.HBM),
            pl.BlockSpec(memory_space=pltpu.MemorySpace.VMEM),
            pl.BlockSpec(memory_space=pltpu.MemorySpace.VMEM),
            pl.BlockSpec(memory_space=pltpu.MemorySpace.VMEM),
        ],
        out_specs=pl.BlockSpec(memory_space=pltpu.MemorySpace.HBM),
        out_shape=jax.ShapeDtypeStruct((n, out_f), jnp.float32),
        scratch_shapes=[pltpu.VMEM((n, out_f), jnp.float32)],
        compiler_params=pltpu.CompilerParams(
            vmem_limit_bytes=63 * 1024 * 1024,
        ),
    )(adj, input, weight, bias2d)
